# baseline (speedup 1.0000x reference)
import functools

import jax
import jax.numpy as jnp
from jax import lax
from jax.experimental import pallas as pl
from jax.experimental.pallas import tpu as pltpu
from jax.experimental.pallas import tpu_sc as plsc

_V = 1000000
_D = 32
_B = 16384
_H = 50
_VBLK = 128


def _wid():
    return lax.axis_index("s") * 2 + lax.axis_index("c")


def _gather_fmt(idx2d, tab_lin):
    mesh = plsc.VectorSubcoreMesh(core_axis_name="c", subcore_axis_name="s")
    n_blocks = 200

    @functools.partial(
        pl.kernel,
        mesh=mesh,
        out_type=jax.ShapeDtypeStruct((_H * 4 * _VBLK * 8 * _VBLK,),
                                      jnp.float32),
        scratch_types=[
            pltpu.VMEM((_H, 512), jnp.int32),
            pltpu.VMEM((_VBLK, _D), jnp.float32),
            pltpu.VMEM((_VBLK, _D), jnp.float32),
            pltpu.VMEM((_D * _VBLK,), jnp.float32),
            pltpu.VMEM((_D * _VBLK,), jnp.float32),
            pltpu.SemaphoreType.DMA,
            pltpu.SemaphoreType.DMA,
            pltpu.SemaphoreType.DMA,
            pltpu.SemaphoreType.DMA,
        ],
        compiler_params=pltpu.CompilerParams(
            use_tc_tiling_on_sc=False, needs_layout_passes=False),
    )
    def k(idx_hbm, tab_hbm, out_hbm, idx_all, g0, g1, t0, t1,
          sg0, sg1, st0, st1):
        i16 = lax.iota(jnp.int32, 16)
        w = _wid()
        gbuf = (g0, g1)
        tbuf = (t0, t1)
        sg = (sg0, sg1)
        st = (st0, st1)

        for hh in range(_H):
            pltpu.sync_copy(idx_hbm.at[hh, pl.ds(w * 512, 512)],
                            idx_all.at[hh])

        def gather_in(t, b):
            pltpu.async_copy(
                tab_hbm.at[idx_all.at[t // 4, pl.ds((t % 4) * _VBLK, _VBLK)]],
                gbuf[b], sg[b])

        def store_out(t, b):
            h = t // 4
            c = (w * 4) + (t % 4)
            for a in range(4):
                pltpu.async_copy(
                    tbuf[b].at[pl.ds(a * 1024, 1024)],
                    out_hbm.at[pl.ds(((h * 4 + a) * _VBLK + c) * 1024, 1024)],
                    st[b])

        def wait_g(b):
            pltpu.make_async_copy(
                out_hbm.at[pl.ds(0, _VBLK * _D)], gbuf[b], sg[b]).wait()

        def wait_s(b):
            for _ in range(4):
                pltpu.make_async_copy(
                    out_hbm.at[pl.ds(0, 1024)], tbuf[b].at[pl.ds(0, 1024)],
                    st[b]).wait()

        def transpose(b):
            base = i16 * _VBLK
            for l in range(_VBLK):
                for kk in range(2):
                    vec = gbuf[b][l, pl.ds(kk * 16, 16)]
                    plsc.store_scatter(
                        tbuf[b], [base + (kk * 16 * _VBLK + l)], vec)

        gather_in(0, 0)
        wait_g(0)
        gather_in(1, 1)
        transpose(0)
        store_out(0, 0)
        wait_g(1)
        gather_in(2, 0)
        transpose(1)
        store_out(1, 1)

        def body(p, carry):
            c0 = p * 2
            wait_g(0)
            wait_s(1)
            gather_in(c0 + 1, 1)
            transpose(0)
            store_out(c0, 0)
            wait_g(1)
            wait_s(0)
            gather_in(c0 + 2, 0)
            transpose(1)
            store_out(c0 + 1, 1)
            return carry

        lax.fori_loop(1, n_blocks // 2 - 1, body, 0)

        cl = n_blocks - 2
        wait_g(0)
        wait_s(1)
        gather_in(cl + 1, 1)
        transpose(0)
        store_out(cl, 0)
        wait_g(1)
        transpose(1)
        store_out(cl + 1, 1)
        wait_s(0)
        wait_s(1)

    return k(idx2d, tab_lin)


def kernel(sentences_idx, table):
    idxT = sentences_idx.astype(jnp.int32).T
    out_flat = _gather_fmt(idxT, table)
    out5 = out_flat.reshape(_H, 4, _VBLK, 8, _VBLK)
    return out5.transpose(2, 4, 0, 1, 3).reshape(_B, _H, _D)
